# Initial kernel scaffold; baseline (speedup 1.0000x reference)
import jax
import jax.numpy as jnp
from jax.experimental import pallas as pl


def kernel(x, edge_index, edge_attr, bases, W_pre, b_pre, W_f1, b_f1, g1, be1, W_f2, b_f2, g2, be2):
    raise NotImplementedError("write your pallas kernel here")


# speedup vs baseline: 2.7420x; 2.7420x over previous
import functools

import jax
import jax.numpy as jnp
from jax import lax
from jax.experimental import pallas as pl
from jax.experimental.pallas import tpu as pltpu
from jax.experimental.pallas import tpu_sc as plsc

_N = 10000
_E = 320000
_D = 128
_GW = 80
_EB = 512
_SUBCORES = 16
_CORES = 2
_NP = 10112
_ROWS_PER_SUB = _NP // _SUBCORES

_mesh = plsc.VectorSubcoreMesh(core_axis_name="core", subcore_axis_name="subcore")


def _gather_sc(x, src_idx):
    @functools.partial(
        pl.kernel,
        mesh=_mesh,
        out_type=jax.ShapeDtypeStruct((_E, _D), jnp.float32),
    )
    def k(x_hbm, i_hbm, o_hbm):
        def body(i_vmem, o_vmem):
            pltpu.sync_copy(x_hbm.at[i_vmem.at[0]], o_vmem)

        pltpu.emit_pipeline(
            body,
            grid=(_E // _GW,),
            in_specs=[pl.BlockSpec((1, _GW), lambda i: (i, 0))],
            out_specs=[pl.BlockSpec((_GW, _D), lambda i: (i, 0))],
            core_axis_name=("core", "subcore"),
            dimension_semantics=(pltpu.PARALLEL,),
        )(i_hbm, o_hbm)

    return k(x, src_idx)


def _scatter_sc(f, dst_idx, zeros):
    @functools.partial(
        pl.kernel,
        mesh=_mesh,
        out_type=jax.ShapeDtypeStruct((_CORES, _NP, _D), jnp.float32),
        scratch_types=[pltpu.VMEM_SHARED((_NP, _D), jnp.float32)],
    )
    def k(f_hbm, i_hbm, z_hbm, o_hbm, acc):
        cid = lax.axis_index("core")
        sid = lax.axis_index("subcore")
        r0 = sid * _ROWS_PER_SUB
        pltpu.sync_copy(z_hbm.at[pl.ds(r0, _ROWS_PER_SUB)],
                        acc.at[pl.ds(r0, _ROWS_PER_SUB)])
        plsc.subcore_barrier()

        def body(f_vmem, i_vmem):
            pltpu.sync_copy(f_vmem, acc.at[i_vmem.at[0]], add=True)

        pltpu.emit_pipeline(
            body,
            grid=(_E // _GW,),
            in_specs=[pl.BlockSpec((_GW, _D), lambda i: (i, 0)),
                      pl.BlockSpec((1, _GW), lambda i: (i, 0))],
            out_specs=[],
            core_axis_name=("core", "subcore"),
            dimension_semantics=(pltpu.PARALLEL,),
        )(f_hbm, i_hbm)

        plsc.subcore_barrier()
        pltpu.sync_copy(acc.at[pl.ds(r0, _ROWS_PER_SUB)],
                        o_hbm.at[cid, pl.ds(r0, _ROWS_PER_SUB)])

    return k(f, dst_idx, zeros)


def _edge_tc(src_x, edge_attr, bases, w_t, b):
    def body(g_ref, ea_ref, ba_ref, w_ref, b_ref, o_ref):
        z = jnp.dot(g_ref[...] + ea_ref[...], w_ref[...],
                    preferred_element_type=jnp.float32) + b_ref[...]
        gelu = 0.5 * z * (1.0 + lax.erf(z * 0.7071067811865476))
        o_ref[...] = gelu * ba_ref[...]

    return pl.pallas_call(
        body,
        grid=(_E // _EB,),
        in_specs=[
            pl.BlockSpec((_EB, _D), lambda i: (i, 0)),
            pl.BlockSpec((_EB, _D), lambda i: (i, 0)),
            pl.BlockSpec((_EB, _D), lambda i: (i, 0)),
            pl.BlockSpec((_D, _D), lambda i: (0, 0)),
            pl.BlockSpec((1, _D), lambda i: (0, 0)),
        ],
        out_specs=pl.BlockSpec((_EB, _D), lambda i: (i, 0)),
        out_shape=jax.ShapeDtypeStruct((_E, _D), jnp.float32),
    )(src_x, edge_attr, bases, w_t, b)


def _node_tc(x, partials, w1_t, b1, g1, be1, w2_t, b2, g2, be2):
    def body(x_ref, p_ref, w1_ref, b1_ref, g1_ref, be1_ref,
             w2_ref, b2_ref, g2_ref, be2_ref, o_ref):
        y = x_ref[...] + p_ref[0, :, :] + p_ref[1, :, :]

        h = jnp.dot(y, w1_ref[...], preferred_element_type=jnp.float32) + b1_ref[...]
        mean = jnp.mean(h, axis=0, keepdims=True)
        var = jnp.mean((h - mean) ** 2, axis=0, keepdims=True)
        h = (h - mean) * lax.rsqrt(var + 1e-5) * g1_ref[...] + be1_ref[...]
        h = jnp.maximum(h, 0.0)

        h = jnp.dot(h, w2_ref[...], preferred_element_type=jnp.float32) + b2_ref[...]
        mean = jnp.mean(h, axis=0, keepdims=True)
        var = jnp.mean((h - mean) ** 2, axis=0, keepdims=True)
        h = (h - mean) * lax.rsqrt(var + 1e-5) * g2_ref[...] + be2_ref[...]
        h = jnp.maximum(h, 0.0)

        o_ref[...] = y + h

    return pl.pallas_call(
        body,
        out_shape=jax.ShapeDtypeStruct((_N, _D), jnp.float32),
    )(x, partials, w1_t, b1, g1, be1, w2_t, b2, g2, be2)


def kernel(x, edge_index, edge_attr, bases, W_pre, b_pre,
           W_f1, b_f1, g1, be1, W_f2, b_f2, g2, be2):
    src_idx = edge_index[0].astype(jnp.int32).reshape(_E // _GW, _GW)
    dst_idx = edge_index[1].astype(jnp.int32).reshape(_E // _GW, _GW)

    src_x = _gather_sc(x, src_idx)
    f = _edge_tc(src_x, edge_attr, bases, W_pre.T, b_pre.reshape(1, _D))
    zeros = jnp.zeros((_NP, _D), jnp.float32)
    partials = _scatter_sc(f, dst_idx, zeros)[:, :_N, :]
    return _node_tc(x, partials, W_f1.T, b_f1.reshape(1, _D),
                    g1.reshape(1, _D), be1.reshape(1, _D),
                    W_f2.T, b_f2.reshape(1, _D),
                    g2.reshape(1, _D), be2.reshape(1, _D))


# speedup vs baseline: 4.1405x; 1.5100x over previous
import functools

import jax
import jax.numpy as jnp
from jax import lax
from jax.experimental import pallas as pl
from jax.experimental.pallas import tpu as pltpu
from jax.experimental.pallas import tpu_sc as plsc

_N = 10000
_E = 320000
_D = 128
_GW = 80
_EB = 2560
_SUBCORES = 16
_CORES = 2
_NP = 10112
_ROWS_PER_SUB = _NP // _SUBCORES

_mesh = plsc.VectorSubcoreMesh(core_axis_name="core", subcore_axis_name="subcore")


def _gather_sc(x, src_idx):
    @functools.partial(
        pl.kernel,
        mesh=_mesh,
        out_type=jax.ShapeDtypeStruct((_E, _D), jnp.float32),
    )
    def k(x_hbm, i_hbm, o_hbm):
        def body(i_vmem, o_vmem):
            pltpu.sync_copy(x_hbm.at[i_vmem.at[0]], o_vmem)

        pltpu.emit_pipeline(
            body,
            grid=(_E // _GW,),
            in_specs=[pl.BlockSpec((1, _GW), lambda i: (i, 0))],
            out_specs=[pl.BlockSpec((_GW, _D), lambda i: (i, 0))],
            core_axis_name=("core", "subcore"),
            dimension_semantics=(pltpu.PARALLEL,),
        )(i_hbm, o_hbm)

    return k(x, src_idx)


def _scatter_sc(f, dst_idx, zeros):
    @functools.partial(
        pl.kernel,
        mesh=_mesh,
        out_type=jax.ShapeDtypeStruct((_CORES, _NP, _D), jnp.float32),
        scratch_types=[pltpu.VMEM_SHARED((_NP, _D), jnp.float32)],
    )
    def k(f_hbm, i_hbm, z_hbm, o_hbm, acc):
        cid = lax.axis_index("core")
        sid = lax.axis_index("subcore")
        r0 = sid * _ROWS_PER_SUB
        pltpu.sync_copy(z_hbm.at[pl.ds(r0, _ROWS_PER_SUB)],
                        acc.at[pl.ds(r0, _ROWS_PER_SUB)])
        plsc.subcore_barrier()

        def body(f_vmem, i_vmem):
            pltpu.sync_copy(f_vmem, acc.at[i_vmem.at[0]], add=True)

        pltpu.emit_pipeline(
            body,
            grid=(_E // _GW,),
            in_specs=[pl.BlockSpec((_GW, _D), lambda i: (i, 0)),
                      pl.BlockSpec((1, _GW), lambda i: (i, 0))],
            out_specs=[],
            core_axis_name=("core", "subcore"),
            dimension_semantics=(pltpu.PARALLEL,),
        )(f_hbm, i_hbm)

        plsc.subcore_barrier()
        pltpu.sync_copy(acc.at[pl.ds(r0, _ROWS_PER_SUB)],
                        o_hbm.at[cid, pl.ds(r0, _ROWS_PER_SUB)])

    return k(f, dst_idx, zeros)


def _edge_tc(src_x, edge_attr, bases, w_t, b):
    def body(g_ref, ea_ref, ba_ref, w_ref, b_ref, o_ref):
        z = jnp.dot(g_ref[...] + ea_ref[...], w_ref[...],
                    preferred_element_type=jnp.float32) + b_ref[...]
        gelu = 0.5 * z * (1.0 + lax.erf(z * 0.7071067811865476))
        o_ref[...] = gelu * ba_ref[...]

    return pl.pallas_call(
        body,
        grid=(_E // _EB,),
        in_specs=[
            pl.BlockSpec((_EB, _D), lambda i: (i, 0)),
            pl.BlockSpec((_EB, _D), lambda i: (i, 0)),
            pl.BlockSpec((_EB, _D), lambda i: (i, 0)),
            pl.BlockSpec((_D, _D), lambda i: (0, 0)),
            pl.BlockSpec((1, _D), lambda i: (0, 0)),
        ],
        out_specs=pl.BlockSpec((_EB, _D), lambda i: (i, 0)),
        out_shape=jax.ShapeDtypeStruct((_E, _D), jnp.float32),
    )(src_x, edge_attr, bases, w_t, b)


def _node_tc(x, partials, w1_t, b1, g1, be1, w2_t, b2, g2, be2):
    def body(x_ref, p_ref, w1_ref, b1_ref, g1_ref, be1_ref,
             w2_ref, b2_ref, g2_ref, be2_ref, o_ref):
        y = x_ref[...] + p_ref[0, :_N, :] + p_ref[1, :_N, :]

        h = jnp.dot(y, w1_ref[...], preferred_element_type=jnp.float32) + b1_ref[...]
        mean = jnp.mean(h, axis=0, keepdims=True)
        var = jnp.mean((h - mean) ** 2, axis=0, keepdims=True)
        h = (h - mean) * lax.rsqrt(var + 1e-5) * g1_ref[...] + be1_ref[...]
        h = jnp.maximum(h, 0.0)

        h = jnp.dot(h, w2_ref[...], preferred_element_type=jnp.float32) + b2_ref[...]
        mean = jnp.mean(h, axis=0, keepdims=True)
        var = jnp.mean((h - mean) ** 2, axis=0, keepdims=True)
        h = (h - mean) * lax.rsqrt(var + 1e-5) * g2_ref[...] + be2_ref[...]
        h = jnp.maximum(h, 0.0)

        o_ref[...] = y + h

    return pl.pallas_call(
        body,
        out_shape=jax.ShapeDtypeStruct((_N, _D), jnp.float32),
    )(x, partials, w1_t, b1, g1, be1, w2_t, b2, g2, be2)


def kernel(x, edge_index, edge_attr, bases, W_pre, b_pre,
           W_f1, b_f1, g1, be1, W_f2, b_f2, g2, be2):
    src_idx = edge_index[0].astype(jnp.int32).reshape(_E // _GW, _GW)
    dst_idx = edge_index[1].astype(jnp.int32).reshape(_E // _GW, _GW)

    src_x = _gather_sc(x, src_idx)
    f = _edge_tc(src_x, edge_attr, bases, W_pre.T, b_pre.reshape(1, _D))
    zeros = jnp.zeros((_NP, _D), jnp.float32)
    partials = _scatter_sc(f, dst_idx, zeros)
    return _node_tc(x, partials, W_f1.T, b_f1.reshape(1, _D),
                    g1.reshape(1, _D), be1.reshape(1, _D),
                    W_f2.T, b_f2.reshape(1, _D),
                    g2.reshape(1, _D), be2.reshape(1, _D))


# speedup vs baseline: 4.4499x; 1.0747x over previous
import functools

import jax
import jax.numpy as jnp
from jax import lax
from jax.experimental import pallas as pl
from jax.experimental.pallas import tpu as pltpu
from jax.experimental.pallas import tpu_sc as plsc

_N = 10000
_E = 320000
_D = 128
_GW = 80
_EB = 2560
_SUBCORES = 16
_CORES = 2
_NP = 10112
_ROWS_PER_SUB = _NP // _SUBCORES

_C = 5
_CW = _E // _C
_GWC = _CW // _GW
_EBC = _CW // _EB

_mesh = plsc.VectorSubcoreMesh(core_axis_name="core", subcore_axis_name="subcore")


def _gather_sc(x, src_idx, ci):
    @functools.partial(
        pl.kernel,
        mesh=_mesh,
        out_type=jax.ShapeDtypeStruct((_CW, _D), jnp.float32),
    )
    def k(x_hbm, i_hbm, o_hbm):
        def body(i_vmem, o_vmem):
            pltpu.sync_copy(x_hbm.at[i_vmem.at[0]], o_vmem)

        pltpu.emit_pipeline(
            body,
            grid=(_GWC,),
            in_specs=[pl.BlockSpec((1, _GW), lambda i, c=ci: (c * _GWC + i, 0))],
            out_specs=[pl.BlockSpec((_GW, _D), lambda i: (i, 0))],
            core_axis_name=("core", "subcore"),
            dimension_semantics=(pltpu.PARALLEL,),
        )(i_hbm, o_hbm)

    return k(x, src_idx)


def _scatter_sc(fs, dst_idx, zeros):
    @functools.partial(
        pl.kernel,
        mesh=_mesh,
        out_type=jax.ShapeDtypeStruct((_CORES, _NP, _D), jnp.float32),
        scratch_types=[pltpu.VMEM_SHARED((_NP, _D), jnp.float32)],
    )
    def k(*refs):
        f_hbms = refs[:_C]
        i_hbm, z_hbm, o_hbm, acc = refs[_C:]
        cid = lax.axis_index("core")
        sid = lax.axis_index("subcore")
        r0 = sid * _ROWS_PER_SUB
        pltpu.sync_copy(z_hbm.at[pl.ds(r0, _ROWS_PER_SUB)],
                        acc.at[pl.ds(r0, _ROWS_PER_SUB)])
        plsc.subcore_barrier()

        def body(f_vmem, i_vmem):
            pltpu.sync_copy(f_vmem, acc.at[i_vmem.at[0]], add=True)

        for ci in range(_C):
            pltpu.emit_pipeline(
                body,
                grid=(_GWC,),
                in_specs=[pl.BlockSpec((_GW, _D), lambda i: (i, 0)),
                          pl.BlockSpec((1, _GW), lambda i, c=ci: (c * _GWC + i, 0))],
                out_specs=[],
                core_axis_name=("core", "subcore"),
                dimension_semantics=(pltpu.PARALLEL,),
            )(f_hbms[ci], i_hbm)

        plsc.subcore_barrier()
        pltpu.sync_copy(acc.at[pl.ds(r0, _ROWS_PER_SUB)],
                        o_hbm.at[cid, pl.ds(r0, _ROWS_PER_SUB)])

    return k(*fs, dst_idx, zeros)


def _edge_tc(src_x_c, edge_attr, bases, w_t, b, ci):
    def body(g_ref, ea_ref, ba_ref, w_ref, b_ref, o_ref):
        z = jnp.dot(g_ref[...] + ea_ref[...], w_ref[...],
                    preferred_element_type=jnp.float32) + b_ref[...]
        gelu = 0.5 * z * (1.0 + lax.erf(z * 0.7071067811865476))
        o_ref[...] = gelu * ba_ref[...]

    return pl.pallas_call(
        body,
        grid=(_EBC,),
        in_specs=[
            pl.BlockSpec((_EB, _D), lambda i: (i, 0)),
            pl.BlockSpec((_EB, _D), lambda i, c=ci: (c * _EBC + i, 0)),
            pl.BlockSpec((_EB, _D), lambda i, c=ci: (c * _EBC + i, 0)),
            pl.BlockSpec((_D, _D), lambda i: (0, 0)),
            pl.BlockSpec((1, _D), lambda i: (0, 0)),
        ],
        out_specs=pl.BlockSpec((_EB, _D), lambda i: (i, 0)),
        out_shape=jax.ShapeDtypeStruct((_CW, _D), jnp.float32),
    )(src_x_c, edge_attr, bases, w_t, b)


def _node_tc(x, partials, w1_t, b1, g1, be1, w2_t, b2, g2, be2):
    def body(x_ref, p_ref, w1_ref, b1_ref, g1_ref, be1_ref,
             w2_ref, b2_ref, g2_ref, be2_ref, o_ref):
        y = x_ref[...] + p_ref[0, :_N, :] + p_ref[1, :_N, :]

        h = jnp.dot(y, w1_ref[...], preferred_element_type=jnp.float32) + b1_ref[...]
        mean = jnp.mean(h, axis=0, keepdims=True)
        var = jnp.mean((h - mean) ** 2, axis=0, keepdims=True)
        h = (h - mean) * lax.rsqrt(var + 1e-5) * g1_ref[...] + be1_ref[...]
        h = jnp.maximum(h, 0.0)

        h = jnp.dot(h, w2_ref[...], preferred_element_type=jnp.float32) + b2_ref[...]
        mean = jnp.mean(h, axis=0, keepdims=True)
        var = jnp.mean((h - mean) ** 2, axis=0, keepdims=True)
        h = (h - mean) * lax.rsqrt(var + 1e-5) * g2_ref[...] + be2_ref[...]
        h = jnp.maximum(h, 0.0)

        o_ref[...] = y + h

    return pl.pallas_call(
        body,
        out_shape=jax.ShapeDtypeStruct((_N, _D), jnp.float32),
    )(x, partials, w1_t, b1, g1, be1, w2_t, b2, g2, be2)


def kernel(x, edge_index, edge_attr, bases, W_pre, b_pre,
           W_f1, b_f1, g1, be1, W_f2, b_f2, g2, be2):
    src_idx = edge_index[0].astype(jnp.int32).reshape(_E // _GW, _GW)
    dst_idx = edge_index[1].astype(jnp.int32).reshape(_E // _GW, _GW)

    w_pre_t = W_pre.T
    b_pre_r = b_pre.reshape(1, _D)

    fs = []
    for ci in range(_C):
        g = _gather_sc(x, src_idx, ci)
        fs.append(_edge_tc(g, edge_attr, bases, w_pre_t, b_pre_r, ci))

    zeros = jnp.zeros((_NP, _D), jnp.float32)
    partials = _scatter_sc(fs, dst_idx, zeros)
    return _node_tc(x, partials, W_f1.T, b_f1.reshape(1, _D),
                    g1.reshape(1, _D), be1.reshape(1, _D),
                    W_f2.T, b_f2.reshape(1, _D),
                    g2.reshape(1, _D), be2.reshape(1, _D))


# speedup vs baseline: 4.5123x; 1.0140x over previous
import functools

import jax
import jax.numpy as jnp
from jax import lax
from jax.experimental import pallas as pl
from jax.experimental.pallas import tpu as pltpu
from jax.experimental.pallas import tpu_sc as plsc

_N = 10000
_E = 320000
_D = 128
_GW = 80
_EB = 2560
_SUBCORES = 16
_CORES = 2
_NP = 10112
_ROWS_PER_SUB = _NP // _SUBCORES

_C = 5
_CW = _E // _C
_GWC = _CW // _GW
_EBC = _CW // _EB

_mesh = plsc.VectorSubcoreMesh(core_axis_name="core", subcore_axis_name="subcore")


_NW = _CORES * _SUBCORES
_WPW = _GWC // _NW
_ROWS_PER_WORKER = _WPW * _GW


def _gather_sc(x, src_idx4, ci):
    @functools.partial(
        pl.kernel,
        mesh=_mesh,
        out_type=jax.ShapeDtypeStruct((_CW, _D), jnp.float32),
        scratch_types=[
            pltpu.VMEM((_WPW, _GW), jnp.int32),
            pltpu.VMEM((_GW, _D), jnp.float32),
            pltpu.VMEM((_GW, _D), jnp.float32),
            pltpu.SemaphoreType.DMA,
            pltpu.SemaphoreType.DMA,
            pltpu.SemaphoreType.DMA,
            pltpu.SemaphoreType.DMA,
        ],
    )
    def k(x_hbm, i_hbm, o_hbm, idx_v, buf0, buf1, gs0, gs1, ws0, ws1):
        cid = lax.axis_index("core")
        sid = lax.axis_index("subcore")
        w = sid * _CORES + cid
        base = w * _ROWS_PER_WORKER
        bufs = (buf0, buf1)
        gsems = (gs0, gs1)
        wsems = (ws0, ws1)

        pltpu.sync_copy(i_hbm.at[ci, w], idx_v)

        gathers = [None] * _WPW
        writes = [None] * _WPW
        gathers[0] = pltpu.async_copy(x_hbm.at[idx_v.at[0]], bufs[0], gsems[0])
        for j in range(_WPW):
            if j + 1 < _WPW:
                if j >= 1:
                    writes[j - 1].wait()
                p = (j + 1) % 2
                gathers[j + 1] = pltpu.async_copy(
                    x_hbm.at[idx_v.at[j + 1]], bufs[p], gsems[p])
            gathers[j].wait()
            writes[j] = pltpu.async_copy(
                bufs[j % 2], o_hbm.at[pl.ds(base + j * _GW, _GW)], wsems[j % 2])
        writes[_WPW - 2].wait()
        writes[_WPW - 1].wait()

    return k(x, src_idx4)


def _scatter_sc(fs, dst_idx, zeros):
    @functools.partial(
        pl.kernel,
        mesh=_mesh,
        out_type=jax.ShapeDtypeStruct((_CORES, _NP, _D), jnp.float32),
        scratch_types=[pltpu.VMEM_SHARED((_NP, _D), jnp.float32)],
    )
    def k(*refs):
        f_hbms = refs[:_C]
        i_hbm, z_hbm, o_hbm, acc = refs[_C:]
        cid = lax.axis_index("core")
        sid = lax.axis_index("subcore")
        r0 = sid * _ROWS_PER_SUB
        pltpu.sync_copy(z_hbm.at[pl.ds(r0, _ROWS_PER_SUB)],
                        acc.at[pl.ds(r0, _ROWS_PER_SUB)])
        plsc.subcore_barrier()

        def body(f_vmem, i_vmem):
            pltpu.sync_copy(f_vmem, acc.at[i_vmem.at[0]], add=True)

        for ci in range(_C):
            pltpu.emit_pipeline(
                body,
                grid=(_GWC,),
                in_specs=[pl.BlockSpec((_GW, _D), lambda i: (i, 0)),
                          pl.BlockSpec((1, _GW), lambda i, c=ci: (c * _GWC + i, 0))],
                out_specs=[],
                core_axis_name=("core", "subcore"),
                dimension_semantics=(pltpu.PARALLEL,),
            )(f_hbms[ci], i_hbm)

        plsc.subcore_barrier()
        pltpu.sync_copy(acc.at[pl.ds(r0, _ROWS_PER_SUB)],
                        o_hbm.at[cid, pl.ds(r0, _ROWS_PER_SUB)])

    return k(*fs, dst_idx, zeros)


def _edge_tc(src_x_c, edge_attr, bases, w_t, b, ci):
    def body(g_ref, ea_ref, ba_ref, w_ref, b_ref, o_ref):
        z = jnp.dot(g_ref[...] + ea_ref[...], w_ref[...],
                    preferred_element_type=jnp.float32) + b_ref[...]
        gelu = 0.5 * z * (1.0 + lax.erf(z * 0.7071067811865476))
        o_ref[...] = gelu * ba_ref[...]

    return pl.pallas_call(
        body,
        grid=(_EBC,),
        in_specs=[
            pl.BlockSpec((_EB, _D), lambda i: (i, 0)),
            pl.BlockSpec((_EB, _D), lambda i, c=ci: (c * _EBC + i, 0)),
            pl.BlockSpec((_EB, _D), lambda i, c=ci: (c * _EBC + i, 0)),
            pl.BlockSpec((_D, _D), lambda i: (0, 0)),
            pl.BlockSpec((1, _D), lambda i: (0, 0)),
        ],
        out_specs=pl.BlockSpec((_EB, _D), lambda i: (i, 0)),
        out_shape=jax.ShapeDtypeStruct((_CW, _D), jnp.float32),
    )(src_x_c, edge_attr, bases, w_t, b)


def _node_tc(x, partials, w1_t, b1, g1, be1, w2_t, b2, g2, be2):
    def body(x_ref, p_ref, w1_ref, b1_ref, g1_ref, be1_ref,
             w2_ref, b2_ref, g2_ref, be2_ref, o_ref):
        y = x_ref[...] + p_ref[0, :_N, :] + p_ref[1, :_N, :]

        h = jnp.dot(y, w1_ref[...], preferred_element_type=jnp.float32) + b1_ref[...]
        mean = jnp.mean(h, axis=0, keepdims=True)
        var = jnp.mean((h - mean) ** 2, axis=0, keepdims=True)
        h = (h - mean) * lax.rsqrt(var + 1e-5) * g1_ref[...] + be1_ref[...]
        h = jnp.maximum(h, 0.0)

        h = jnp.dot(h, w2_ref[...], preferred_element_type=jnp.float32) + b2_ref[...]
        mean = jnp.mean(h, axis=0, keepdims=True)
        var = jnp.mean((h - mean) ** 2, axis=0, keepdims=True)
        h = (h - mean) * lax.rsqrt(var + 1e-5) * g2_ref[...] + be2_ref[...]
        h = jnp.maximum(h, 0.0)

        o_ref[...] = y + h

    return pl.pallas_call(
        body,
        out_shape=jax.ShapeDtypeStruct((_N, _D), jnp.float32),
    )(x, partials, w1_t, b1, g1, be1, w2_t, b2, g2, be2)


def kernel(x, edge_index, edge_attr, bases, W_pre, b_pre,
           W_f1, b_f1, g1, be1, W_f2, b_f2, g2, be2):
    src_idx4 = edge_index[0].astype(jnp.int32).reshape(_C, _NW, _WPW, _GW)
    dst_idx = edge_index[1].astype(jnp.int32).reshape(_E // _GW, _GW)

    w_pre_t = W_pre.T
    b_pre_r = b_pre.reshape(1, _D)

    fs = []
    for ci in range(_C):
        g = _gather_sc(x, src_idx4, ci)
        fs.append(_edge_tc(g, edge_attr, bases, w_pre_t, b_pre_r, ci))

    zeros = jnp.zeros((_NP, _D), jnp.float32)
    partials = _scatter_sc(fs, dst_idx, zeros)
    return _node_tc(x, partials, W_f1.T, b_f1.reshape(1, _D),
                    g1.reshape(1, _D), be1.reshape(1, _D),
                    W_f2.T, b_f2.reshape(1, _D),
                    g2.reshape(1, _D), be2.reshape(1, _D))
